# Initial kernel scaffold; baseline (speedup 1.0000x reference)
import jax
import jax.numpy as jnp
from jax.experimental import pallas as pl


def kernel(fine_xyz, coarse_xyz, fine_feat, coarse_feat, W1, g1, b1, W2, g2, b2):
    raise NotImplementedError("write your pallas kernel here")


# speedup vs baseline: 22.1105x; 22.1105x over previous
import functools

import jax
import jax.numpy as jnp
import numpy as np
from jax.experimental import pallas as pl
from jax.experimental.pallas import tpu as pltpu

_B, _N, _S, _K = 2, 8192, 2048, 3
_FINE_C, _COARSE_C, _OUT_C = 64, 128, 128
_IN_C = _COARSE_C + _FINE_C
_MID = max(_IN_C // 2, _OUT_C)
_GROUPS = 8
_TILE_N = 512
_EPS = 1e-5


def _stage1_kernel(fxyz_ref, cxyzt_ref, ffeat_ref, cfeat_ref, w1_ref, o_ref):
    f = fxyz_ref[0]
    ct = cxyzt_ref[0]
    fx, fy, fz = f[:, 0:1], f[:, 1:2], f[:, 2:3]
    cx, cy, cz = ct[0:1, :], ct[1:2, :], ct[2:3, :]

    dx = fx - cx
    dy = fy - cy
    dz = fz - cz
    d2e = (dx * dx + dy * dy) + dz * dz

    def b16(v):
        return v.astype(jnp.bfloat16).astype(jnp.float32)
    prod = (b16(fx) * b16(cx) + b16(fy) * b16(cy)) + b16(fz) * b16(cz)
    fn2 = (fx * fx + fy * fy) + fz * fz
    cn2 = (cx * cx + cy * cy) + cz * cz
    d2n = fn2 + cn2 - 2.0 * prod

    m1 = jnp.min(d2n, axis=1, keepdims=True)
    d2a = jnp.where(d2n == m1, jnp.inf, d2n)
    m2 = jnp.min(d2a, axis=1, keepdims=True)
    d2b = jnp.where(d2a == m2, jnp.inf, d2a)
    m3 = jnp.min(d2b, axis=1, keepdims=True)

    w = jnp.where(d2n <= m3, 1.0 / jnp.maximum(d2e, 1e-8), 0.0)
    wnorm = 1.0 / jnp.sum(w, axis=1, keepdims=True)

    interp = jax.lax.dot_general(w, cfeat_ref[0], (((1,), (0,)), ((), ())),
                                 preferred_element_type=jnp.float32, precision=jax.lax.Precision.HIGHEST)
    interp = interp * wnorm
    ff = ffeat_ref[0]
    w1 = w1_ref[...]
    x1 = jax.lax.dot_general(ff, w1[:, :_FINE_C], (((1,), (1,)), ((), ())),
                             preferred_element_type=jnp.float32, precision=jax.lax.Precision.HIGHEST)
    x1 = x1 + jax.lax.dot_general(interp, w1[:, _FINE_C:], (((1,), (1,)), ((), ())),
                                  preferred_element_type=jnp.float32, precision=jax.lax.Precision.HIGHEST)
    o_ref[0] = x1


def _stage2_kernel(x_ref, gmat_ref, g1_ref, b1_ref, w2_ref, g2_ref, b2_ref, o_ref):
    gmat = gmat_ref[...]

    def gn_relu(x, gamma, beta, n_per_group):
        s = jnp.sum(x, axis=0, keepdims=True)
        sq = jnp.sum(x * x, axis=0, keepdims=True)
        gs = jax.lax.dot_general(s, gmat, (((1,), (0,)), ((), ())),
                                 preferred_element_type=jnp.float32, precision=jax.lax.Precision.HIGHEST)
        gq = jax.lax.dot_general(sq, gmat, (((1,), (0,)), ((), ())),
                                 preferred_element_type=jnp.float32, precision=jax.lax.Precision.HIGHEST)
        mean = gs / n_per_group
        var = gq / n_per_group - mean * mean
        inv = jax.lax.rsqrt(var + _EPS)
        mean_c = jax.lax.dot_general(mean, gmat, (((1,), (1,)), ((), ())),
                                     preferred_element_type=jnp.float32, precision=jax.lax.Precision.HIGHEST)
        inv_c = jax.lax.dot_general(inv, gmat, (((1,), (1,)), ((), ())),
                                    preferred_element_type=jnp.float32, precision=jax.lax.Precision.HIGHEST)
        return jnp.maximum((x - mean_c) * (inv_c * gamma) + beta, 0.0)

    x = x_ref[0]
    h = gn_relu(x, g1_ref[...], b1_ref[...], float(_N * (_MID // _GROUPS)))
    x2 = jax.lax.dot_general(h, w2_ref[...], (((1,), (1,)), ((), ())),
                             preferred_element_type=jnp.float32, precision=jax.lax.Precision.HIGHEST)
    o_ref[0] = gn_relu(x2, g2_ref[...], b2_ref[...], float(_N * (_OUT_C // _GROUPS)))


@jax.jit
def kernel(fine_xyz, coarse_xyz, fine_feat, coarse_feat, W1, g1, b1, W2, g2, b2):
    B, N, _ = fine_xyz.shape
    S = coarse_xyz.shape[1]
    cxyzt = jnp.transpose(coarse_xyz, (0, 2, 1))

    nt = N // _TILE_N
    x1 = pl.pallas_call(
        _stage1_kernel,
        grid=(B, nt),
        in_specs=[
            pl.BlockSpec((1, _TILE_N, 3), lambda b, i: (b, i, 0)),
            pl.BlockSpec((1, 3, S), lambda b, i: (b, 0, 0)),
            pl.BlockSpec((1, _TILE_N, _FINE_C), lambda b, i: (b, i, 0)),
            pl.BlockSpec((1, S, _COARSE_C), lambda b, i: (b, 0, 0)),
            pl.BlockSpec((_MID, _IN_C), lambda b, i: (0, 0)),
        ],
        out_specs=pl.BlockSpec((1, _TILE_N, _MID), lambda b, i: (b, i, 0)),
        out_shape=jax.ShapeDtypeStruct((B, N, _MID), jnp.float32),
        compiler_params=pltpu.CompilerParams(
            dimension_semantics=("parallel", "parallel")),
    )(fine_xyz, cxyzt, fine_feat, coarse_feat, W1)

    gmat = (jnp.arange(_MID)[:, None] // (_MID // _GROUPS)
            == jnp.arange(_GROUPS)[None, :]).astype(jnp.float32)
    out = pl.pallas_call(
        _stage2_kernel,
        grid=(B,),
        in_specs=[
            pl.BlockSpec((1, N, _MID), lambda b: (b, 0, 0)),
            pl.BlockSpec((_MID, _GROUPS), lambda b: (0, 0)),
            pl.BlockSpec((1, _MID), lambda b: (0, 0)),
            pl.BlockSpec((1, _MID), lambda b: (0, 0)),
            pl.BlockSpec((_OUT_C, _MID), lambda b: (0, 0)),
            pl.BlockSpec((1, _OUT_C), lambda b: (0, 0)),
            pl.BlockSpec((1, _OUT_C), lambda b: (0, 0)),
        ],
        out_specs=pl.BlockSpec((1, N, _OUT_C), lambda b: (b, 0, 0)),
        out_shape=jax.ShapeDtypeStruct((B, N, _OUT_C), jnp.float32),
        compiler_params=pltpu.CompilerParams(
            dimension_semantics=("parallel",)),
    )(x1, gmat, g1.reshape(1, _MID), b1.reshape(1, _MID), W2,
      g2.reshape(1, _OUT_C), b2.reshape(1, _OUT_C))
    return out


# speedup vs baseline: 30.4670x; 1.3779x over previous
import functools

import jax
import jax.numpy as jnp
import numpy as np
from jax.experimental import pallas as pl
from jax.experimental.pallas import tpu as pltpu

_B, _N, _S, _K = 2, 8192, 2048, 3
_FINE_C, _COARSE_C, _OUT_C = 64, 128, 128
_IN_C = _COARSE_C + _FINE_C
_MID = max(_IN_C // 2, _OUT_C)
_GROUPS = 8
_TILE_N = 512
_EPS = 1e-5


def _stage1_kernel(fxyz_ref, cxyzt_ref, ffeat_ref, cfeat_ref, w1_ref, o_ref):
    f = fxyz_ref[0]
    ct = cxyzt_ref[0]
    fx, fy, fz = f[:, 0:1], f[:, 1:2], f[:, 2:3]
    cx, cy, cz = ct[0:1, :], ct[1:2, :], ct[2:3, :]

    dx = fx - cx
    dy = fy - cy
    dz = fz - cz
    d2e = (dx * dx + dy * dy) + dz * dz

    def b16(v):
        return v.astype(jnp.bfloat16).astype(jnp.float32)
    prod = (b16(fx) * b16(cx) + b16(fy) * b16(cy)) + b16(fz) * b16(cz)
    fn2 = (fx * fx + fy * fy) + fz * fz
    cn2 = (cx * cx + cy * cy) + cz * cz
    d2n = fn2 + cn2 - 2.0 * prod

    m1 = jnp.min(d2n, axis=1, keepdims=True)
    d2a = jnp.where(d2n == m1, jnp.inf, d2n)
    m2 = jnp.min(d2a, axis=1, keepdims=True)
    d2b = jnp.where(d2a == m2, jnp.inf, d2a)
    m3 = jnp.min(d2b, axis=1, keepdims=True)

    w = jnp.where(d2n <= m3, 1.0 / jnp.maximum(d2e, 1e-8), 0.0)
    wnorm = 1.0 / jnp.sum(w, axis=1, keepdims=True)

    interp = jax.lax.dot_general(w, cfeat_ref[0], (((1,), (0,)), ((), ())),
                                 preferred_element_type=jnp.float32)
    interp = interp * wnorm
    ff = ffeat_ref[0]
    w1 = w1_ref[...]
    x1 = jax.lax.dot_general(ff, w1[:, :_FINE_C], (((1,), (1,)), ((), ())),
                             preferred_element_type=jnp.float32, precision=jax.lax.Precision.HIGHEST)
    x1 = x1 + jax.lax.dot_general(interp, w1[:, _FINE_C:], (((1,), (1,)), ((), ())),
                                  preferred_element_type=jnp.float32, precision=jax.lax.Precision.HIGHEST)
    o_ref[0] = x1


def _stage2_kernel(x_ref, gmat_ref, g1_ref, b1_ref, w2_ref, g2_ref, b2_ref, o_ref):
    gmat = gmat_ref[...]

    def gn_relu(x, gamma, beta, n_per_group):
        s = jnp.sum(x, axis=0, keepdims=True)
        sq = jnp.sum(x * x, axis=0, keepdims=True)
        gs = jax.lax.dot_general(s, gmat, (((1,), (0,)), ((), ())),
                                 preferred_element_type=jnp.float32, precision=jax.lax.Precision.HIGHEST)
        gq = jax.lax.dot_general(sq, gmat, (((1,), (0,)), ((), ())),
                                 preferred_element_type=jnp.float32, precision=jax.lax.Precision.HIGHEST)
        mean = gs / n_per_group
        var = gq / n_per_group - mean * mean
        inv = jax.lax.rsqrt(var + _EPS)
        mean_c = jax.lax.dot_general(mean, gmat, (((1,), (1,)), ((), ())),
                                     preferred_element_type=jnp.float32, precision=jax.lax.Precision.HIGHEST)
        inv_c = jax.lax.dot_general(inv, gmat, (((1,), (1,)), ((), ())),
                                    preferred_element_type=jnp.float32, precision=jax.lax.Precision.HIGHEST)
        return jnp.maximum((x - mean_c) * (inv_c * gamma) + beta, 0.0)

    x = x_ref[0]
    h = gn_relu(x, g1_ref[...], b1_ref[...], float(_N * (_MID // _GROUPS)))
    x2 = jax.lax.dot_general(h, w2_ref[...], (((1,), (1,)), ((), ())),
                             preferred_element_type=jnp.float32, precision=jax.lax.Precision.HIGHEST)
    o_ref[0] = gn_relu(x2, g2_ref[...], b2_ref[...], float(_N * (_OUT_C // _GROUPS)))


@jax.jit
def kernel(fine_xyz, coarse_xyz, fine_feat, coarse_feat, W1, g1, b1, W2, g2, b2):
    B, N, _ = fine_xyz.shape
    S = coarse_xyz.shape[1]
    cxyzt = jnp.transpose(coarse_xyz, (0, 2, 1))

    nt = N // _TILE_N
    x1 = pl.pallas_call(
        _stage1_kernel,
        grid=(B, nt),
        in_specs=[
            pl.BlockSpec((1, _TILE_N, 3), lambda b, i: (b, i, 0)),
            pl.BlockSpec((1, 3, S), lambda b, i: (b, 0, 0)),
            pl.BlockSpec((1, _TILE_N, _FINE_C), lambda b, i: (b, i, 0)),
            pl.BlockSpec((1, S, _COARSE_C), lambda b, i: (b, 0, 0)),
            pl.BlockSpec((_MID, _IN_C), lambda b, i: (0, 0)),
        ],
        out_specs=pl.BlockSpec((1, _TILE_N, _MID), lambda b, i: (b, i, 0)),
        out_shape=jax.ShapeDtypeStruct((B, N, _MID), jnp.float32),
        compiler_params=pltpu.CompilerParams(
            dimension_semantics=("parallel", "parallel")),
    )(fine_xyz, cxyzt, fine_feat, coarse_feat, W1)

    gmat = (jnp.arange(_MID)[:, None] // (_MID // _GROUPS)
            == jnp.arange(_GROUPS)[None, :]).astype(jnp.float32)
    out = pl.pallas_call(
        _stage2_kernel,
        grid=(B,),
        in_specs=[
            pl.BlockSpec((1, N, _MID), lambda b: (b, 0, 0)),
            pl.BlockSpec((_MID, _GROUPS), lambda b: (0, 0)),
            pl.BlockSpec((1, _MID), lambda b: (0, 0)),
            pl.BlockSpec((1, _MID), lambda b: (0, 0)),
            pl.BlockSpec((_OUT_C, _MID), lambda b: (0, 0)),
            pl.BlockSpec((1, _OUT_C), lambda b: (0, 0)),
            pl.BlockSpec((1, _OUT_C), lambda b: (0, 0)),
        ],
        out_specs=pl.BlockSpec((1, N, _OUT_C), lambda b: (b, 0, 0)),
        out_shape=jax.ShapeDtypeStruct((B, N, _OUT_C), jnp.float32),
        compiler_params=pltpu.CompilerParams(
            dimension_semantics=("parallel",)),
    )(x1, gmat, g1.reshape(1, _MID), b1.reshape(1, _MID), W2,
      g2.reshape(1, _OUT_C), b2.reshape(1, _OUT_C))
    return out


# speedup vs baseline: 38.2157x; 1.2543x over previous
import functools

import jax
import jax.numpy as jnp
import numpy as np
from jax.experimental import pallas as pl
from jax.experimental.pallas import tpu as pltpu

_B, _N, _S, _K = 2, 8192, 2048, 3
_FINE_C, _COARSE_C, _OUT_C = 64, 128, 128
_IN_C = _COARSE_C + _FINE_C
_MID = max(_IN_C // 2, _OUT_C)
_GROUPS = 8
_TILE_N = 512
_EPS = 1e-5


def _stage1_kernel(fxyz_ref, cxyzt_ref, ffeat_ref, cfeat_ref, w1_ref, o_ref):
    f = fxyz_ref[0]
    ct = cxyzt_ref[0]
    fx, fy, fz = f[:, 0:1], f[:, 1:2], f[:, 2:3]
    cx, cy, cz = ct[0:1, :], ct[1:2, :], ct[2:3, :]

    dx = fx - cx
    dy = fy - cy
    dz = fz - cz
    d2e = (dx * dx + dy * dy) + dz * dz

    def b16(v):
        return v.astype(jnp.bfloat16).astype(jnp.float32)
    prod = (b16(fx) * b16(cx) + b16(fy) * b16(cy)) + b16(fz) * b16(cz)
    fn2 = (fx * fx + fy * fy) + fz * fz
    cn2 = (cx * cx + cy * cy) + cz * cz
    d2n = fn2 + cn2 - 2.0 * prod

    m1 = jnp.min(d2n, axis=1, keepdims=True)
    d2a = jnp.where(d2n == m1, jnp.inf, d2n)
    m2 = jnp.min(d2a, axis=1, keepdims=True)
    d2b = jnp.where(d2a == m2, jnp.inf, d2a)
    m3 = jnp.min(d2b, axis=1, keepdims=True)

    w = jnp.where(d2n <= m3, 1.0 / jnp.maximum(d2e, 1e-8), 0.0)
    wnorm = 1.0 / jnp.sum(w, axis=1, keepdims=True)

    interp = jax.lax.dot_general(w, cfeat_ref[0], (((1,), (0,)), ((), ())),
                                 preferred_element_type=jnp.float32)
    interp = interp * wnorm
    ff = ffeat_ref[0]
    w1 = w1_ref[...]
    x1 = jax.lax.dot_general(ff, w1[:, :_FINE_C], (((1,), (1,)), ((), ())),
                             preferred_element_type=jnp.float32)
    x1 = x1 + jax.lax.dot_general(interp, w1[:, _FINE_C:], (((1,), (1,)), ((), ())),
                                  preferred_element_type=jnp.float32)
    o_ref[0] = x1


def _stage2_kernel(x_ref, gmat_ref, g1_ref, b1_ref, w2_ref, g2_ref, b2_ref, o_ref):
    gmat = gmat_ref[...]

    def gn_relu(x, gamma, beta, n_per_group):
        s = jnp.sum(x, axis=0, keepdims=True)
        sq = jnp.sum(x * x, axis=0, keepdims=True)
        gs = jax.lax.dot_general(s, gmat, (((1,), (0,)), ((), ())),
                                 preferred_element_type=jnp.float32)
        gq = jax.lax.dot_general(sq, gmat, (((1,), (0,)), ((), ())),
                                 preferred_element_type=jnp.float32)
        mean = gs / n_per_group
        var = gq / n_per_group - mean * mean
        inv = jax.lax.rsqrt(var + _EPS)
        mean_c = jax.lax.dot_general(mean, gmat, (((1,), (1,)), ((), ())),
                                     preferred_element_type=jnp.float32)
        inv_c = jax.lax.dot_general(inv, gmat, (((1,), (1,)), ((), ())),
                                    preferred_element_type=jnp.float32)
        return jnp.maximum((x - mean_c) * (inv_c * gamma) + beta, 0.0)

    x = x_ref[0]
    h = gn_relu(x, g1_ref[...], b1_ref[...], float(_N * (_MID // _GROUPS)))
    x2 = jax.lax.dot_general(h, w2_ref[...], (((1,), (1,)), ((), ())),
                             preferred_element_type=jnp.float32)
    o_ref[0] = gn_relu(x2, g2_ref[...], b2_ref[...], float(_N * (_OUT_C // _GROUPS)))


@jax.jit
def kernel(fine_xyz, coarse_xyz, fine_feat, coarse_feat, W1, g1, b1, W2, g2, b2):
    B, N, _ = fine_xyz.shape
    S = coarse_xyz.shape[1]
    cxyzt = jnp.transpose(coarse_xyz, (0, 2, 1))

    nt = N // _TILE_N
    x1 = pl.pallas_call(
        _stage1_kernel,
        grid=(B, nt),
        in_specs=[
            pl.BlockSpec((1, _TILE_N, 3), lambda b, i: (b, i, 0)),
            pl.BlockSpec((1, 3, S), lambda b, i: (b, 0, 0)),
            pl.BlockSpec((1, _TILE_N, _FINE_C), lambda b, i: (b, i, 0)),
            pl.BlockSpec((1, S, _COARSE_C), lambda b, i: (b, 0, 0)),
            pl.BlockSpec((_MID, _IN_C), lambda b, i: (0, 0)),
        ],
        out_specs=pl.BlockSpec((1, _TILE_N, _MID), lambda b, i: (b, i, 0)),
        out_shape=jax.ShapeDtypeStruct((B, N, _MID), jnp.float32),
        compiler_params=pltpu.CompilerParams(
            dimension_semantics=("parallel", "parallel")),
    )(fine_xyz, cxyzt, fine_feat, coarse_feat, W1)

    gmat = (jnp.arange(_MID)[:, None] // (_MID // _GROUPS)
            == jnp.arange(_GROUPS)[None, :]).astype(jnp.float32)
    out = pl.pallas_call(
        _stage2_kernel,
        grid=(B,),
        in_specs=[
            pl.BlockSpec((1, N, _MID), lambda b: (b, 0, 0)),
            pl.BlockSpec((_MID, _GROUPS), lambda b: (0, 0)),
            pl.BlockSpec((1, _MID), lambda b: (0, 0)),
            pl.BlockSpec((1, _MID), lambda b: (0, 0)),
            pl.BlockSpec((_OUT_C, _MID), lambda b: (0, 0)),
            pl.BlockSpec((1, _OUT_C), lambda b: (0, 0)),
            pl.BlockSpec((1, _OUT_C), lambda b: (0, 0)),
        ],
        out_specs=pl.BlockSpec((1, N, _OUT_C), lambda b: (b, 0, 0)),
        out_shape=jax.ShapeDtypeStruct((B, N, _OUT_C), jnp.float32),
        compiler_params=pltpu.CompilerParams(
            dimension_semantics=("parallel",)),
    )(x1, gmat, g1.reshape(1, _MID), b1.reshape(1, _MID), W2,
      g2.reshape(1, _OUT_C), b2.reshape(1, _OUT_C))
    return out


# speedup vs baseline: 38.4619x; 1.0064x over previous
import functools

import jax
import jax.numpy as jnp
import numpy as np
from jax.experimental import pallas as pl
from jax.experimental.pallas import tpu as pltpu

_B, _N, _S, _K = 2, 8192, 2048, 3
_FINE_C, _COARSE_C, _OUT_C = 64, 128, 128
_IN_C = _COARSE_C + _FINE_C
_MID = max(_IN_C // 2, _OUT_C)
_GROUPS = 8
_TILE_N = 1024
_EPS = 1e-5


def _stage1_kernel(fxyz_ref, cxyzt_ref, ffeat_ref, cfeat_ref, w1_ref, o_ref):
    f = fxyz_ref[0]
    ct = cxyzt_ref[0]
    fx, fy, fz = f[:, 0:1], f[:, 1:2], f[:, 2:3]
    cx, cy, cz = ct[0:1, :], ct[1:2, :], ct[2:3, :]

    dx = fx - cx
    dy = fy - cy
    dz = fz - cz
    d2e = (dx * dx + dy * dy) + dz * dz

    def b16(v):
        return v.astype(jnp.bfloat16).astype(jnp.float32)
    prod = (b16(fx) * b16(cx) + b16(fy) * b16(cy)) + b16(fz) * b16(cz)
    fn2 = (fx * fx + fy * fy) + fz * fz
    cn2 = (cx * cx + cy * cy) + cz * cz
    d2n = fn2 + cn2 - 2.0 * prod

    m1 = jnp.min(d2n, axis=1, keepdims=True)
    d2a = jnp.where(d2n == m1, jnp.inf, d2n)
    m2 = jnp.min(d2a, axis=1, keepdims=True)
    d2b = jnp.where(d2a == m2, jnp.inf, d2a)
    m3 = jnp.min(d2b, axis=1, keepdims=True)

    w = jnp.where(d2n <= m3, 1.0 / jnp.maximum(d2e, 1e-8), 0.0)
    wnorm = 1.0 / jnp.sum(w, axis=1, keepdims=True)

    interp = jax.lax.dot_general(w, cfeat_ref[0], (((1,), (0,)), ((), ())),
                                 preferred_element_type=jnp.float32)
    interp = interp * wnorm
    ff = ffeat_ref[0]
    w1 = w1_ref[...]
    x1 = jax.lax.dot_general(ff, w1[:, :_FINE_C], (((1,), (1,)), ((), ())),
                             preferred_element_type=jnp.float32)
    x1 = x1 + jax.lax.dot_general(interp, w1[:, _FINE_C:], (((1,), (1,)), ((), ())),
                                  preferred_element_type=jnp.float32)
    o_ref[0] = x1


def _stage2_kernel(x_ref, gmat_ref, g1_ref, b1_ref, w2_ref, g2_ref, b2_ref, o_ref):
    gmat = gmat_ref[...]

    def gn_relu(x, gamma, beta, n_per_group):
        s = jnp.sum(x, axis=0, keepdims=True)
        sq = jnp.sum(x * x, axis=0, keepdims=True)
        gs = jax.lax.dot_general(s, gmat, (((1,), (0,)), ((), ())),
                                 preferred_element_type=jnp.float32)
        gq = jax.lax.dot_general(sq, gmat, (((1,), (0,)), ((), ())),
                                 preferred_element_type=jnp.float32)
        mean = gs / n_per_group
        var = gq / n_per_group - mean * mean
        inv = jax.lax.rsqrt(var + _EPS)
        mean_c = jax.lax.dot_general(mean, gmat, (((1,), (1,)), ((), ())),
                                     preferred_element_type=jnp.float32)
        inv_c = jax.lax.dot_general(inv, gmat, (((1,), (1,)), ((), ())),
                                    preferred_element_type=jnp.float32)
        return jnp.maximum((x - mean_c) * (inv_c * gamma) + beta, 0.0)

    x = x_ref[0]
    h = gn_relu(x, g1_ref[...], b1_ref[...], float(_N * (_MID // _GROUPS)))
    x2 = jax.lax.dot_general(h, w2_ref[...], (((1,), (1,)), ((), ())),
                             preferred_element_type=jnp.float32)
    o_ref[0] = gn_relu(x2, g2_ref[...], b2_ref[...], float(_N * (_OUT_C // _GROUPS)))


@jax.jit
def kernel(fine_xyz, coarse_xyz, fine_feat, coarse_feat, W1, g1, b1, W2, g2, b2):
    B, N, _ = fine_xyz.shape
    S = coarse_xyz.shape[1]
    cxyzt = jnp.transpose(coarse_xyz, (0, 2, 1))

    nt = N // _TILE_N
    x1 = pl.pallas_call(
        _stage1_kernel,
        grid=(B, nt),
        in_specs=[
            pl.BlockSpec((1, _TILE_N, 3), lambda b, i: (b, i, 0)),
            pl.BlockSpec((1, 3, S), lambda b, i: (b, 0, 0)),
            pl.BlockSpec((1, _TILE_N, _FINE_C), lambda b, i: (b, i, 0)),
            pl.BlockSpec((1, S, _COARSE_C), lambda b, i: (b, 0, 0)),
            pl.BlockSpec((_MID, _IN_C), lambda b, i: (0, 0)),
        ],
        out_specs=pl.BlockSpec((1, _TILE_N, _MID), lambda b, i: (b, i, 0)),
        out_shape=jax.ShapeDtypeStruct((B, N, _MID), jnp.float32),
        compiler_params=pltpu.CompilerParams(
            dimension_semantics=("parallel", "parallel")),
    )(fine_xyz, cxyzt, fine_feat, coarse_feat, W1)

    gmat = (jnp.arange(_MID)[:, None] // (_MID // _GROUPS)
            == jnp.arange(_GROUPS)[None, :]).astype(jnp.float32)
    out = pl.pallas_call(
        _stage2_kernel,
        grid=(B,),
        in_specs=[
            pl.BlockSpec((1, N, _MID), lambda b: (b, 0, 0)),
            pl.BlockSpec((_MID, _GROUPS), lambda b: (0, 0)),
            pl.BlockSpec((1, _MID), lambda b: (0, 0)),
            pl.BlockSpec((1, _MID), lambda b: (0, 0)),
            pl.BlockSpec((_OUT_C, _MID), lambda b: (0, 0)),
            pl.BlockSpec((1, _OUT_C), lambda b: (0, 0)),
            pl.BlockSpec((1, _OUT_C), lambda b: (0, 0)),
        ],
        out_specs=pl.BlockSpec((1, N, _OUT_C), lambda b: (b, 0, 0)),
        out_shape=jax.ShapeDtypeStruct((B, N, _OUT_C), jnp.float32),
        compiler_params=pltpu.CompilerParams(
            dimension_semantics=("parallel",)),
    )(x1, gmat, g1.reshape(1, _MID), b1.reshape(1, _MID), W2,
      g2.reshape(1, _OUT_C), b2.reshape(1, _OUT_C))
    return out
